# baseline (speedup 1.0000x reference)
import functools

import jax
import jax.numpy as jnp
from jax import lax
from jax.experimental import pallas as pl
from jax.experimental.pallas import tpu as pltpu
from jax.experimental.pallas import tpu_sc as plsc

B, T, D = 64, 576, 768
E, R = 8, 192
SPP = 8
_CHUNKS = B // 16


def _fused_kernel(tok_ref, gw_ref, gb_ref, wd_ref, bd_ref, wu_ref, bu_ref,
                  out_ref, logits_ref, imp_ref, load_ref):
    g = pl.program_id(0)
    lane = jax.lax.broadcasted_iota(jnp.int32, (1, E), 1)
    imp_acc = jnp.zeros((1, E), jnp.float32)
    load_acc = jnp.zeros((1, E), jnp.float32)
    for i in range(SPP):
        x = tok_ref[i]

        pooled = jnp.mean(x, axis=0, keepdims=True)
        logits = jnp.dot(pooled, gw_ref[...],
                         preferred_element_type=jnp.float32) + gb_ref[...]
        m = jnp.max(logits, axis=-1, keepdims=True)
        ex = jnp.exp(logits - m)
        probs = ex / jnp.sum(ex, axis=-1, keepdims=True)
        top1 = jnp.max(probs)
        sel = jnp.argmax(probs, axis=-1)[0].astype(jnp.int32)

        wd = wd_ref[sel]
        wu = wu_ref[sel]
        bd = bd_ref[sel]
        bu = bu_ref[sel]
        xb = x.astype(jnp.bfloat16)
        h = jnp.dot(xb, wd, preferred_element_type=jnp.float32) + bd
        h = jax.nn.gelu(h.astype(jnp.bfloat16))
        y = jnp.dot(h, wu, preferred_element_type=jnp.float32) + bu
        out_ref[i] = top1 * (x + y)

        logits_ref[i] = logits
        onehot = (lane == sel).astype(jnp.float32)
        imp_acc += onehot * top1
        load_acc += onehot * (1.0 / B)

    @pl.when(g == 0)
    def _init():
        imp_ref[...] = jnp.zeros_like(imp_ref)
        load_ref[...] = jnp.zeros_like(load_ref)

    imp_ref[...] += imp_acc
    load_ref[...] += load_acc


def _sc_router(logits_hbm, sel_hbm, ew_hbm, lg_v, sel_v, ew_v):
    cid = lax.axis_index("c")
    sid = lax.axis_index("s")

    @pl.when((cid == 0) & (sid == 0))
    def _body():
        pltpu.sync_copy(logits_hbm, lg_v)
        zero16 = jnp.zeros((16,), jnp.float32)

        for k in range(_CHUNKS):
            cols = [lg_v[pl.ds(e * B + 16 * k, 16)] for e in range(E)]
            m = cols[0]
            for e in range(1, E):
                m = jnp.maximum(m, cols[e])
            s = zero16
            for e in range(E):
                s = s + jnp.exp(cols[e] - m)
            top1 = 1.0 / s
            selv = jnp.zeros((16,), jnp.int32)
            for e in reversed(range(E)):
                selv = jnp.where(cols[e] == m, e, selv)

            sel_v[pl.ds(16 * k, 16)] = selv
            for e in range(E):
                ew_v[pl.ds(e * B + 16 * k, 16)] = jnp.where(
                    selv == e, top1, 0.0)

        pltpu.sync_copy(sel_v, sel_hbm)
        pltpu.sync_copy(ew_v, ew_hbm)


@jax.jit
def kernel(tokens, spatial_shape, gate_W, gate_b, W_down, b_down, W_up, b_up):
    del spatial_shape
    gb2 = gate_b.reshape(1, E)
    bd3 = b_down.reshape(E, 1, R)
    bu3 = b_up.reshape(E, 1, D)
    wd_bf = W_down.astype(jnp.bfloat16)
    wu_bf = W_up.astype(jnp.bfloat16)

    out, logits3, imp, load = pl.pallas_call(
        _fused_kernel,
        grid=(B // SPP,),
        in_specs=[
            pl.BlockSpec((SPP, T, D), lambda b: (b, 0, 0)),
            pl.BlockSpec((D, E), lambda b: (0, 0)),
            pl.BlockSpec((1, E), lambda b: (0, 0)),
            pl.BlockSpec((E, D, R), lambda b: (0, 0, 0)),
            pl.BlockSpec((E, 1, R), lambda b: (0, 0, 0)),
            pl.BlockSpec((E, R, D), lambda b: (0, 0, 0)),
            pl.BlockSpec((E, 1, D), lambda b: (0, 0, 0)),
        ],
        out_specs=[
            pl.BlockSpec((SPP, T, D), lambda b: (b, 0, 0)),
            pl.BlockSpec((SPP, 1, E), lambda b: (b, 0, 0)),
            pl.BlockSpec((1, E), lambda b: (0, 0)),
            pl.BlockSpec((1, E), lambda b: (0, 0)),
        ],
        out_shape=[
            jax.ShapeDtypeStruct((B, T, D), jnp.float32),
            jax.ShapeDtypeStruct((B, 1, E), jnp.float32),
            jax.ShapeDtypeStruct((1, E), jnp.float32),
            jax.ShapeDtypeStruct((1, E), jnp.float32),
        ],
        compiler_params=pltpu.CompilerParams(
            vmem_limit_bytes=100 * 1024 * 1024),
    )(tokens, gate_W, gb2, wd_bf, bd3, wu_bf, bu3)

    router_logits = logits3.reshape(B, E)

    sc = functools.partial(
        pl.kernel,
        mesh=plsc.VectorSubcoreMesh(core_axis_name="c", subcore_axis_name="s"),
        out_type=[
            jax.ShapeDtypeStruct((B,), jnp.int32),
            jax.ShapeDtypeStruct((E * B,), jnp.float32),
        ],
        scratch_types=[
            pltpu.VMEM((E * B,), jnp.float32),
            pltpu.VMEM((B,), jnp.int32),
            pltpu.VMEM((E * B,), jnp.float32),
        ],
    )
    sel, ew_flat = sc(_sc_router)(router_logits.T.reshape(E * B))

    selected_experts = sel.reshape(B, 1)
    expert_weights = ew_flat.reshape(E, B).T
    return (out, router_logits, selected_experts, expert_weights,
            imp.reshape(E), load.reshape(E))


# speedup vs baseline: 1.0142x; 1.0142x over previous
import functools

import jax
import jax.numpy as jnp
from jax import lax
from jax.experimental import pallas as pl
from jax.experimental.pallas import tpu as pltpu
from jax.experimental.pallas import tpu_sc as plsc

B, T, D = 64, 576, 768
E, R = 8, 192
SPP = 4
_CHUNKS = B // 16


def _fused_kernel(tok_ref, gw_ref, gb_ref, wd_ref, bd_ref, wu_ref, bu_ref,
                  out_ref, logits_ref, imp_ref, load_ref):
    g = pl.program_id(0)
    lane = jax.lax.broadcasted_iota(jnp.int32, (1, E), 1)
    imp_acc = jnp.zeros((1, E), jnp.float32)
    load_acc = jnp.zeros((1, E), jnp.float32)
    for i in range(SPP):
        x = tok_ref[i]

        pooled = jnp.mean(x, axis=0, keepdims=True)
        logits = jnp.dot(pooled, gw_ref[...],
                         preferred_element_type=jnp.float32) + gb_ref[...]
        m = jnp.max(logits, axis=-1, keepdims=True)
        ex = jnp.exp(logits - m)
        probs = ex / jnp.sum(ex, axis=-1, keepdims=True)
        top1 = jnp.max(probs)
        sel = jnp.argmax(probs, axis=-1)[0].astype(jnp.int32)

        wd = wd_ref[sel]
        wu = wu_ref[sel]
        bd = bd_ref[sel]
        bu = bu_ref[sel]
        xb = x.astype(jnp.bfloat16)
        h = jnp.dot(xb, wd, preferred_element_type=jnp.float32) + bd
        h = jax.nn.gelu(h.astype(jnp.bfloat16))
        y = jnp.dot(h, wu, preferred_element_type=jnp.float32) + bu
        out_ref[i] = top1 * (x + y)

        logits_ref[i] = logits
        onehot = (lane == sel).astype(jnp.float32)
        imp_acc += onehot * top1
        load_acc += onehot * (1.0 / B)

    @pl.when(g == 0)
    def _init():
        imp_ref[...] = jnp.zeros_like(imp_ref)
        load_ref[...] = jnp.zeros_like(load_ref)

    imp_ref[...] += imp_acc
    load_ref[...] += load_acc


def _sc_router(logits_hbm, sel_hbm, ew_hbm, lg_v, sel_v, ew_v):
    cid = lax.axis_index("c")
    sid = lax.axis_index("s")

    @pl.when((cid == 0) & (sid == 0))
    def _body():
        pltpu.sync_copy(logits_hbm, lg_v)
        zero16 = jnp.zeros((16,), jnp.float32)

        for k in range(_CHUNKS):
            cols = [lg_v[pl.ds(e * B + 16 * k, 16)] for e in range(E)]
            m = cols[0]
            for e in range(1, E):
                m = jnp.maximum(m, cols[e])
            s = zero16
            for e in range(E):
                s = s + jnp.exp(cols[e] - m)
            top1 = 1.0 / s
            selv = jnp.zeros((16,), jnp.int32)
            for e in reversed(range(E)):
                selv = jnp.where(cols[e] == m, e, selv)

            sel_v[pl.ds(16 * k, 16)] = selv
            for e in range(E):
                ew_v[pl.ds(e * B + 16 * k, 16)] = jnp.where(
                    selv == e, top1, 0.0)

        pltpu.sync_copy(sel_v, sel_hbm)
        pltpu.sync_copy(ew_v, ew_hbm)


@jax.jit
def kernel(tokens, spatial_shape, gate_W, gate_b, W_down, b_down, W_up, b_up):
    del spatial_shape
    gb2 = gate_b.reshape(1, E)
    bd3 = b_down.reshape(E, 1, R)
    bu3 = b_up.reshape(E, 1, D)
    wd_bf = W_down.astype(jnp.bfloat16)
    wu_bf = W_up.astype(jnp.bfloat16)

    out, logits3, imp, load = pl.pallas_call(
        _fused_kernel,
        grid=(B // SPP,),
        in_specs=[
            pl.BlockSpec((SPP, T, D), lambda b: (b, 0, 0)),
            pl.BlockSpec((D, E), lambda b: (0, 0)),
            pl.BlockSpec((1, E), lambda b: (0, 0)),
            pl.BlockSpec((E, D, R), lambda b: (0, 0, 0)),
            pl.BlockSpec((E, 1, R), lambda b: (0, 0, 0)),
            pl.BlockSpec((E, R, D), lambda b: (0, 0, 0)),
            pl.BlockSpec((E, 1, D), lambda b: (0, 0, 0)),
        ],
        out_specs=[
            pl.BlockSpec((SPP, T, D), lambda b: (b, 0, 0)),
            pl.BlockSpec((SPP, 1, E), lambda b: (b, 0, 0)),
            pl.BlockSpec((1, E), lambda b: (0, 0)),
            pl.BlockSpec((1, E), lambda b: (0, 0)),
        ],
        out_shape=[
            jax.ShapeDtypeStruct((B, T, D), jnp.float32),
            jax.ShapeDtypeStruct((B, 1, E), jnp.float32),
            jax.ShapeDtypeStruct((1, E), jnp.float32),
            jax.ShapeDtypeStruct((1, E), jnp.float32),
        ],
        compiler_params=pltpu.CompilerParams(
            vmem_limit_bytes=100 * 1024 * 1024),
    )(tokens, gate_W, gb2, wd_bf, bd3, wu_bf, bu3)

    router_logits = logits3.reshape(B, E)

    sc = functools.partial(
        pl.kernel,
        mesh=plsc.VectorSubcoreMesh(core_axis_name="c", subcore_axis_name="s"),
        out_type=[
            jax.ShapeDtypeStruct((B,), jnp.int32),
            jax.ShapeDtypeStruct((E * B,), jnp.float32),
        ],
        scratch_types=[
            pltpu.VMEM((E * B,), jnp.float32),
            pltpu.VMEM((B,), jnp.int32),
            pltpu.VMEM((E * B,), jnp.float32),
        ],
    )
    sel, ew_flat = sc(_sc_router)(router_logits.T.reshape(E * B))

    selected_experts = sel.reshape(B, 1)
    expert_weights = ew_flat.reshape(E, B).T
    return (out, router_logits, selected_experts, expert_weights,
            imp.reshape(E), load.reshape(E))
